# baseline (speedup 1.0000x reference)
import jax
import jax.numpy as jnp
import numpy as np
from jax import lax
from jax.experimental import pallas as pl
from jax.experimental.pallas import tpu as pltpu
from jax.experimental.pallas import tpu_sc as plsc

N_LEVELS = 16
F = 2
LOG2_T = 19
T = 1 << LOG2_T
BASE_RES = 16
PER_LEVEL_SCALE = 1.5
N_POINTS = 1048576
RES = [int(np.floor(BASE_RES * (PER_LEVEL_SCALE ** l))) for l in range(N_LEVELS)]
DENSE = [(r + 1) ** 2 <= T for r in RES]
PRIME_I32 = int(np.int32(np.uint32(2654435761).view(np.int32)))
MASK = T - 1

NW = 32
PTS_PER_W = N_POINTS // NW
B = 512
CHUNKS = PTS_PER_W // B
G = B // 16
PBLK = B // 128

N_RES_LEVELS = 6
_off = 0
RES_OFF = []
for _l in range(N_RES_LEVELS):
    _nr = (RES[_l] + 1) ** 2
    RES_OFF.append((_l, _off, _nr))
    _off += (_nr + 7) // 8 * 8
RES_ROWS = _off

STAGED = list(range(N_RES_LEVELS, N_LEVELS))


def _body(x0_hbm, x1_hbm, tab1d_hbm, out_hbm,
          tbl_v, x0_v, x1_v, idx0_v, idx1_v, w_v, feat0_v, feat1_v,
          out_v, sem):
    wid = lax.axis_index("s") * 2 + lax.axis_index("c")

    for (l, off, nr) in RES_OFF:
        nrp = (nr + 7) // 8 * 8
        pltpu.sync_copy(tab1d_hbm.at[pl.ds(l * T * 2, 2 * nrp)],
                        tbl_v.at[pl.ds(2 * off, 2 * nrp)])

    base0 = wid * PTS_PER_W

    def corners(xv, yv, r):
        posx = xv * float(r)
        posy = yv * float(r)
        ix = posx.astype(jnp.int32)
        iy = posy.astype(jnp.int32)
        fx = posx - ix.astype(jnp.float32)
        fy = posy - iy.astype(jnp.float32)
        wx1 = fx
        wx0 = 1.0 - fx
        wy1 = fy
        wy0 = 1.0 - fy
        ws = (wx0 * wy0, wx0 * wy1, wx1 * wy0, wx1 * wy1)
        return ix, iy, ws

    def dense_idx(ix, iy, r, base):
        b00 = iy * (r + 1) + ix + base
        return (b00, b00 + (r + 1), b00 + 1, b00 + r + 2)

    def hash_idx(ix, iy, base):
        m0 = iy * PRIME_I32
        m1 = m0 + PRIME_I32
        ix1 = ix + 1
        return (((ix ^ m0) & MASK) + base,
                ((ix ^ m1) & MASK) + base,
                ((ix1 ^ m0) & MASK) + base,
                ((ix1 ^ m1) & MASK) + base)

    def out_store(l, g, acc0, acc1):
        s = g * 16
        fb = (2 * l) // 8
        f0 = (2 * l) % 8
        pb = s // 128
        pi = s - pb * 128
        out_v[fb, pb, f0, pl.ds(pi, 16)] = acc0
        out_v[fb, pb, f0 + 1, pl.ds(pi, 16)] = acc1

    def chunk_body(ci, carry):
        pbase = base0 + ci * B
        pltpu.sync_copy(x0_hbm.at[pl.ds(pbase, B)], x0_v)
        pltpu.sync_copy(x1_hbm.at[pl.ds(pbase, B)], x1_v)

        for (l, off, nr) in RES_OFF:
            r = RES[l]

            def res_group(g, c, l=l, off=off, r=r):
                s = g * 16
                xv = x0_v[pl.ds(s, 16)]
                yv = x1_v[pl.ds(s, 16)]
                ix, iy, ws = corners(xv, yv, r)
                ids = dense_idx(ix, iy, r, off)
                acc0 = jnp.zeros((16,), jnp.float32)
                acc1 = jnp.zeros((16,), jnp.float32)
                for c4 in range(4):
                    iv = ids[c4] + ids[c4]
                    f0 = plsc.load_gather(tbl_v, [iv])
                    f1 = plsc.load_gather(tbl_v, [iv + 1])
                    acc0 = acc0 + ws[c4] * f0
                    acc1 = acc1 + ws[c4] * f1
                out_store(l, g, acc0, acc1)
                return c

            lax.fori_loop(0, G, res_group, 0, unroll=False)

        for l in STAGED:
            r = RES[l]
            dense = DENSE[l]

            def idx_group(g, c, l=l, r=r, dense=dense):
                s = g * 16
                xv = x0_v[pl.ds(s, 16)]
                yv = x1_v[pl.ds(s, 16)]
                ix, iy, ws = corners(xv, yv, r)
                if dense:
                    ids = dense_idx(ix, iy, r, l * T)
                else:
                    ids = hash_idx(ix, iy, l * T)
                for c4 in range(4):
                    a = ids[c4] + ids[c4]
                    idx0_v[pl.ds(c4 * B + s, 16)] = a
                    idx1_v[pl.ds(c4 * B + s, 16)] = a + 1
                    w_v[c4, pl.ds(s, 16)] = ws[c4]
                return c

            lax.fori_loop(0, G, idx_group, 0, unroll=False)

            cp0 = pltpu.async_copy(tab1d_hbm.at[idx0_v], feat0_v, sem)
            cp1 = pltpu.async_copy(tab1d_hbm.at[idx1_v], feat1_v, sem)
            cp0.wait()
            cp1.wait()

            def acc_group(g, c, l=l):
                s = g * 16
                acc0 = jnp.zeros((16,), jnp.float32)
                acc1 = jnp.zeros((16,), jnp.float32)
                for c4 in range(4):
                    f0 = feat0_v[pl.ds(c4 * B + s, 16)]
                    f1 = feat1_v[pl.ds(c4 * B + s, 16)]
                    w = w_v[c4, pl.ds(s, 16)]
                    acc0 = acc0 + w * f0
                    acc1 = acc1 + w * f1
                out_store(l, g, acc0, acc1)
                return c

            lax.fori_loop(0, G, acc_group, 0, unroll=False)

        pb0 = pbase // 128
        for fb in range(4):
            pltpu.sync_copy(out_v.at[fb], out_hbm.at[fb, pl.ds(pb0, PBLK)])
        return carry

    lax.fori_loop(0, CHUNKS, chunk_body, 0, unroll=False)


@jax.jit
def kernel(x, table):
    x0 = x[:, 0] + 0.0
    x1 = x[:, 1] + 0.0
    tab1d = table.reshape(-1)
    mesh = plsc.VectorSubcoreMesh(core_axis_name="c", subcore_axis_name="s")
    f = pl.kernel(
        _body,
        out_type=jax.ShapeDtypeStruct((4, N_POINTS // 128, 8, 128), jnp.float32),
        mesh=mesh,
        compiler_params=pltpu.CompilerParams(
            needs_layout_passes=False, use_tc_tiling_on_sc=False),
        scratch_types=[
            pltpu.VMEM((RES_ROWS * F,), jnp.float32),
            pltpu.VMEM((B,), jnp.float32),
            pltpu.VMEM((B,), jnp.float32),
            pltpu.VMEM((4 * B,), jnp.int32),
            pltpu.VMEM((4 * B,), jnp.int32),
            pltpu.VMEM((4, B), jnp.float32),
            pltpu.VMEM((4 * B,), jnp.float32),
            pltpu.VMEM((4 * B,), jnp.float32),
            pltpu.VMEM((4, PBLK, 8, 128), jnp.float32),
            pltpu.SemaphoreType.DMA,
        ],
    )
    out4 = f(x0, x1, tab1d)
    return out4.transpose(1, 3, 0, 2).reshape(N_POINTS, 2 * N_LEVELS)


# speedup vs baseline: 1.0885x; 1.0885x over previous
import jax
import jax.numpy as jnp
import numpy as np
from jax import lax
from jax.experimental import pallas as pl
from jax.experimental.pallas import tpu as pltpu
from jax.experimental.pallas import tpu_sc as plsc

N_LEVELS = 16
F = 2
LOG2_T = 19
T = 1 << LOG2_T
BASE_RES = 16
PER_LEVEL_SCALE = 1.5
N_POINTS = 1048576
RES = [int(np.floor(BASE_RES * (PER_LEVEL_SCALE ** l))) for l in range(N_LEVELS)]
DENSE = [(r + 1) ** 2 <= T for r in RES]
PRIME_I32 = int(np.int32(np.uint32(2654435761).view(np.int32)))
MASK = T - 1

NW = 32
PTS_PER_W = N_POINTS // NW
B = 512
CHUNKS = PTS_PER_W // B
G = B // 16
PBLK = B // 128

N_RES_LEVELS = 6
_off = 0
RES_OFF = []
for _l in range(N_RES_LEVELS):
    _nr = (RES[_l] + 1) ** 2
    RES_OFF.append((_l, _off, _nr))
    _off += (_nr + 7) // 8 * 8
RES_ROWS = _off

STAGED = list(range(N_RES_LEVELS, N_LEVELS))


def _body(x0_hbm, x1_hbm, tab1d_hbm, out_hbm,
          tbl_v, x0_v, x1_v,
          idx0a_v, idx1a_v, idx0b_v, idx1b_v,
          wa_v, wb_v, feat0a_v, feat1a_v, feat0b_v, feat1b_v,
          out_v, sema, semb):
    wid = lax.axis_index("s") * 2 + lax.axis_index("c")

    for (l, off, nr) in RES_OFF:
        nrp = (nr + 7) // 8 * 8
        pltpu.sync_copy(tab1d_hbm.at[pl.ds(l * T * 2, 2 * nrp)],
                        tbl_v.at[pl.ds(2 * off, 2 * nrp)])

    base0 = wid * PTS_PER_W

    def corners(xv, yv, r):
        posx = xv * float(r)
        posy = yv * float(r)
        ix = posx.astype(jnp.int32)
        iy = posy.astype(jnp.int32)
        fx = posx - ix.astype(jnp.float32)
        fy = posy - iy.astype(jnp.float32)
        wx1 = fx
        wx0 = 1.0 - fx
        wy1 = fy
        wy0 = 1.0 - fy
        ws = (wx0 * wy0, wx0 * wy1, wx1 * wy0, wx1 * wy1)
        return ix, iy, ws

    def dense_idx(ix, iy, r, base):
        b00 = iy * (r + 1) + ix + base
        return (b00, b00 + (r + 1), b00 + 1, b00 + r + 2)

    def hash_idx(ix, iy, base):
        m0 = iy * PRIME_I32
        m1 = m0 + PRIME_I32
        ix1 = ix + 1
        return (((ix ^ m0) & MASK) + base,
                ((ix ^ m1) & MASK) + base,
                ((ix1 ^ m0) & MASK) + base,
                ((ix1 ^ m1) & MASK) + base)

    def out_store(l, g, acc0, acc1):
        s = g * 16
        fb = (2 * l) // 8
        f0 = (2 * l) % 8
        pb = s // 128
        pi = s - pb * 128
        out_v[fb, pb, f0, pl.ds(pi, 16)] = acc0
        out_v[fb, pb, f0 + 1, pl.ds(pi, 16)] = acc1

    bufs = [(idx0a_v, idx1a_v, wa_v, feat0a_v, feat1a_v, sema),
            (idx0b_v, idx1b_v, wb_v, feat0b_v, feat1b_v, semb)]

    def fill_and_start(l):
        p = (l - N_RES_LEVELS) % 2
        idx0_v, idx1_v, w_v, feat0_v, feat1_v, sem = bufs[p]
        r = RES[l]
        dense = DENSE[l]

        def idx_group(g, c, l=l, r=r, dense=dense):
            s = g * 16
            xv = x0_v[pl.ds(s, 16)]
            yv = x1_v[pl.ds(s, 16)]
            ix, iy, ws = corners(xv, yv, r)
            if dense:
                ids = dense_idx(ix, iy, r, l * T)
            else:
                ids = hash_idx(ix, iy, l * T)
            for c4 in range(4):
                a = ids[c4] + ids[c4]
                idx0_v[pl.ds(c4 * B + s, 16)] = a
                idx1_v[pl.ds(c4 * B + s, 16)] = a + 1
                w_v[c4, pl.ds(s, 16)] = ws[c4]
            return c

        lax.fori_loop(0, G, idx_group, 0, unroll=False)
        cp0 = pltpu.async_copy(tab1d_hbm.at[idx0_v], feat0_v, sem)
        cp1 = pltpu.async_copy(tab1d_hbm.at[idx1_v], feat1_v, sem)
        return cp0, cp1

    def acc_level(l):
        p = (l - N_RES_LEVELS) % 2
        _, _, w_v, feat0_v, feat1_v, _ = bufs[p]

        def acc_group(g, c, l=l):
            s = g * 16
            acc0 = jnp.zeros((16,), jnp.float32)
            acc1 = jnp.zeros((16,), jnp.float32)
            for c4 in range(4):
                f0 = feat0_v[pl.ds(c4 * B + s, 16)]
                f1 = feat1_v[pl.ds(c4 * B + s, 16)]
                w = w_v[c4, pl.ds(s, 16)]
                acc0 = acc0 + w * f0
                acc1 = acc1 + w * f1
            out_store(l, g, acc0, acc1)
            return c

        lax.fori_loop(0, G, acc_group, 0, unroll=False)

    def chunk_body(ci, carry):
        pbase = base0 + ci * B
        pltpu.sync_copy(x0_hbm.at[pl.ds(pbase, B)], x0_v)
        pltpu.sync_copy(x1_hbm.at[pl.ds(pbase, B)], x1_v)

        cps = {STAGED[0]: fill_and_start(STAGED[0]),
               STAGED[1]: fill_and_start(STAGED[1])}

        for (l, off, nr) in RES_OFF:
            r = RES[l]

            def res_group(g, c, l=l, off=off, r=r):
                s = g * 16
                xv = x0_v[pl.ds(s, 16)]
                yv = x1_v[pl.ds(s, 16)]
                ix, iy, ws = corners(xv, yv, r)
                ids = dense_idx(ix, iy, r, off)
                acc0 = jnp.zeros((16,), jnp.float32)
                acc1 = jnp.zeros((16,), jnp.float32)
                for c4 in range(4):
                    iv = ids[c4] + ids[c4]
                    f0 = plsc.load_gather(tbl_v, [iv])
                    f1 = plsc.load_gather(tbl_v, [iv + 1])
                    acc0 = acc0 + ws[c4] * f0
                    acc1 = acc1 + ws[c4] * f1
                out_store(l, g, acc0, acc1)
                return c

            lax.fori_loop(0, G, res_group, 0, unroll=False)

        for l in STAGED[2:]:
            cp0, cp1 = cps.pop(l - 2)
            cp0.wait()
            cp1.wait()
            acc_level(l - 2)
            cps[l] = fill_and_start(l)
        for l in STAGED[-2:]:
            cp0, cp1 = cps.pop(l)
            cp0.wait()
            cp1.wait()
            acc_level(l)

        pb0 = pbase // 128
        for fb in range(4):
            pltpu.sync_copy(out_v.at[fb], out_hbm.at[fb, pl.ds(pb0, PBLK)])
        return carry

    lax.fori_loop(0, CHUNKS, chunk_body, 0, unroll=False)


@jax.jit
def kernel(x, table):
    x0 = x[:, 0] + 0.0
    x1 = x[:, 1] + 0.0
    tab1d = table.reshape(-1)
    mesh = plsc.VectorSubcoreMesh(core_axis_name="c", subcore_axis_name="s")
    f = pl.kernel(
        _body,
        out_type=jax.ShapeDtypeStruct((4, N_POINTS // 128, 8, 128), jnp.float32),
        mesh=mesh,
        compiler_params=pltpu.CompilerParams(
            needs_layout_passes=False, use_tc_tiling_on_sc=False),
        scratch_types=[
            pltpu.VMEM((RES_ROWS * F,), jnp.float32),
            pltpu.VMEM((B,), jnp.float32),
            pltpu.VMEM((B,), jnp.float32),
            pltpu.VMEM((4 * B,), jnp.int32),
            pltpu.VMEM((4 * B,), jnp.int32),
            pltpu.VMEM((4 * B,), jnp.int32),
            pltpu.VMEM((4 * B,), jnp.int32),
            pltpu.VMEM((4, B), jnp.float32),
            pltpu.VMEM((4, B), jnp.float32),
            pltpu.VMEM((4 * B,), jnp.float32),
            pltpu.VMEM((4 * B,), jnp.float32),
            pltpu.VMEM((4 * B,), jnp.float32),
            pltpu.VMEM((4 * B,), jnp.float32),
            pltpu.VMEM((4, PBLK, 8, 128), jnp.float32),
            pltpu.SemaphoreType.DMA,
            pltpu.SemaphoreType.DMA,
        ],
    )
    out4 = f(x0, x1, tab1d)
    return out4.transpose(1, 3, 0, 2).reshape(N_POINTS, 2 * N_LEVELS)
